# baseline (speedup 1.0000x reference)
import functools

import jax
import jax.numpy as jnp
from jax import lax
from jax.experimental import pallas as pl
from jax.experimental.pallas import tpu as pltpu
from jax.experimental.pallas import tpu_sc as plsc

_N = 10000
_D = 128
_E = 320000
_NW = 32
_EPW = _E // _NW
_G = 128
_NGF = _EPW // _G
_TB = _NGF * _G


def _pack_bf16_halves(y):
    yb = lax.bitcast_convert_type(y, jnp.int32)
    r = yb + 0x7FFF + (lax.shift_right_logical(yb, 16) & 1)
    lo = lax.shift_right_logical(r[:, : _D // 2], 16)
    hi = r[:, _D // 2:] & jnp.int32(-65536)
    return lo | hi


def _mlp_both_block(eu_ref, w1_ref, b1_ref, ev_ref, w2_ref, b2_ref,
                    u_ref, v_ref):
    yu = jnp.dot(eu_ref[...], w1_ref[...], preferred_element_type=jnp.float32)
    u_ref[...] = _pack_bf16_halves(jnp.maximum(yu + b1_ref[...], 0.0))
    yv = jnp.dot(ev_ref[...], w2_ref[...], preferred_element_type=jnp.float32)
    v_ref[...] = _pack_bf16_halves(jnp.maximum(yv + b2_ref[...], 0.0))


def _transform_both(eu, w1, b1, ev, w2, b2, bl=2000):
    n, d = eu.shape
    row_spec = pl.BlockSpec((bl, d), lambda i: (i, 0))
    w_spec = pl.BlockSpec((d, d), lambda i: (0, 0))
    b_spec = pl.BlockSpec((1, d), lambda i: (0, 0))
    pk_spec = pl.BlockSpec((bl, d // 2), lambda i: (i, 0))
    return pl.pallas_call(
        _mlp_both_block,
        grid=(n // bl,),
        in_specs=[row_spec, w_spec, b_spec, row_spec, w_spec, b_spec],
        out_specs=(pk_spec, pk_spec),
        out_shape=(jax.ShapeDtypeStruct((n, d // 2), jnp.int32),
                   jax.ShapeDtypeStruct((n, d // 2), jnp.int32)),
    )(eu, w1, b1.reshape(1, d), ev, w2, b2.reshape(1, d))


def _edge_values(u_tab, vneg_tab, edge_flat, ev):
    mesh = plsc.VectorSubcoreMesh(core_axis_name="c", subcore_axis_name="s")

    @functools.partial(
        pl.kernel,
        mesh=mesh,
        out_type=jax.ShapeDtypeStruct((_E,), jnp.float32),
        compiler_params=pltpu.CompilerParams(
            needs_layout_passes=False, use_tc_tiling_on_sc=False),
        scratch_types=[
            pltpu.VMEM((_EPW,), jnp.int32),
            pltpu.VMEM((_EPW,), jnp.int32),
            pltpu.VMEM((_EPW,), jnp.float32),
            pltpu.VMEM((_EPW,), jnp.float32),
            pltpu.VMEM((3, _G, _D // 2), jnp.int32),
            pltpu.VMEM((3, _G, _D // 2), jnp.int32),
            pltpu.SemaphoreType.DMA,
            pltpu.SemaphoreType.DMA,
            pltpu.SemaphoreType.DMA,
            pltpu.SemaphoreType.DMA,
            pltpu.SemaphoreType.DMA,
            pltpu.SemaphoreType.DMA,
        ],
    )
    def body(u_hbm, v_hbm, edge_hbm, ev_hbm, out_hbm,
             src_v, dst_v, ev_v, out_v, u_rows, v_rows,
             su0, su1, su2, sv0, sv1, sv2):
        wid = lax.axis_index("s") * 2 + lax.axis_index("c")
        base = wid * _EPW
        pltpu.sync_copy(edge_hbm.at[pl.ds(base, _EPW)], src_v)

        lanes = lax.iota(jnp.int32, 16)
        sem_u = (su0, su1, su2)
        sem_v = (sv0, sv1, sv2)

        def issue(g, b, n=_G):
            pltpu.async_copy(u_hbm.at[src_v.at[pl.ds(g * _G, n)]],
                             u_rows.at[b].at[pl.ds(0, n)], sem_u[b])
            pltpu.async_copy(v_hbm.at[dst_v.at[pl.ds(g * _G, n)]],
                             v_rows.at[b].at[pl.ds(0, n)], sem_v[b])

        def wait(g, b, n=_G):
            pltpu.make_async_copy(u_hbm.at[src_v.at[pl.ds(g * _G, n)]],
                                  u_rows.at[b].at[pl.ds(0, n)],
                                  sem_u[b]).wait()
            pltpu.make_async_copy(v_hbm.at[dst_v.at[pl.ds(g * _G, n)]],
                                  v_rows.at[b].at[pl.ds(0, n)],
                                  sem_v[b]).wait()

        def sub16(b, eb, sgb):
            d2 = jnp.zeros((16,), jnp.float32)
            for k in range(16):
                e = sgb + k
                acc = jnp.zeros((16,), jnp.float32)
                for j in range(_D // 32):
                    uw = u_rows[b, e, pl.ds(j * 16, 16)]
                    vw = v_rows[b, e, pl.ds(j * 16, 16)]
                    du = (plsc.bitcast(uw, jnp.bfloat16)
                          - plsc.bitcast(vw, jnp.bfloat16))
                    lo, hi = plsc.unpack(
                        du, format=plsc.PackFormat.INTERLEAVED)
                    acc = acc + lo * lo + hi * hi
                d2 = jnp.where(lanes == k, jnp.sum(acc), d2)
            d2c = jnp.maximum(d2, 1e-30)
            bi = lax.bitcast_convert_type(d2c, jnp.int32)
            bi = 0x5F3759DF - lax.shift_right_arithmetic(bi, 1)
            y = lax.bitcast_convert_type(bi, jnp.float32)
            for _ in range(3):
                y = y * (1.5 - 0.5 * d2c * y * y)
            dist = d2 * y
            sim = jnp.exp(dist)
            sig = 1.0 / (1.0 + jnp.exp(-sim))
            out_v[pl.ds(eb, 16)] = ev_v[pl.ds(eb, 16)] * sig

        def compute(g, b):
            gb = g * _G

            def subgroup(sg, c):
                sub16(b, gb + sg * 16, sg * 16)
                return c

            lax.fori_loop(0, _G // 16, subgroup, 0)

        pltpu.sync_copy(edge_hbm.at[pl.ds(_E + base, _EPW)], dst_v)
        issue(0, 0)
        pltpu.sync_copy(ev_hbm.at[pl.ds(base, _EPW)], ev_v)
        issue(1, 1)

        def outer(tt, carry):
            g0 = tt * 3
            for k in range(3):
                g = g0 + k
                issue(g + 2, (k + 2) % 3)
                wait(g, k)
                compute(g, k)
            return carry

        lax.fori_loop(0, (_NGF - 3) // 3, outer, 0)
        issue(_NGF - 1, 2)
        wait(_NGF - 3, 0)
        compute(_NGF - 3, 0)
        issue(_NGF, 0, n=16)
        wait(_NGF - 2, 1)
        compute(_NGF - 2, 1)
        wait(_NGF - 1, 2)
        compute(_NGF - 1, 2)
        wait(_NGF, 0, n=16)
        sub16(0, _TB, 0)

        pltpu.sync_copy(out_v, out_hbm.at[pl.ds(base, _EPW)])

    return body(u_tab, vneg_tab, edge_flat, ev)


def kernel(Eu, Ev, W1, b1, W2, b2, edge_index, edge_val):
    u, v = _transform_both(Eu, W1, b1, Ev, W2, b2)
    return _edge_values(u, v, edge_index.reshape(2 * _E), edge_val)


# speedup vs baseline: 1.0132x; 1.0132x over previous
import functools

import jax
import jax.numpy as jnp
from jax import lax
from jax.experimental import pallas as pl
from jax.experimental.pallas import tpu as pltpu
from jax.experimental.pallas import tpu_sc as plsc

_N = 10000
_D = 128
_E = 320000
_NW = 32
_EPW = _E // _NW
_G = 96
_NGF = _EPW // _G
_TB = _NGF * _G


def _pack_bf16_halves(y):
    yb = lax.bitcast_convert_type(y, jnp.int32)
    r = yb + 0x7FFF + (lax.shift_right_logical(yb, 16) & 1)
    lo = lax.shift_right_logical(r[:, : _D // 2], 16)
    hi = r[:, _D // 2:] & jnp.int32(-65536)
    return lo | hi


def _mlp_both_block(eu_ref, w1_ref, b1_ref, ev_ref, w2_ref, b2_ref,
                    u_ref, v_ref):
    yu = jnp.dot(eu_ref[...], w1_ref[...], preferred_element_type=jnp.float32)
    u_ref[...] = _pack_bf16_halves(jnp.maximum(yu + b1_ref[...], 0.0))
    yv = jnp.dot(ev_ref[...], w2_ref[...], preferred_element_type=jnp.float32)
    v_ref[...] = _pack_bf16_halves(jnp.maximum(yv + b2_ref[...], 0.0))


def _transform_both(eu, w1, b1, ev, w2, b2, bl=2000):
    n, d = eu.shape
    row_spec = pl.BlockSpec((bl, d), lambda i: (i, 0))
    w_spec = pl.BlockSpec((d, d), lambda i: (0, 0))
    b_spec = pl.BlockSpec((1, d), lambda i: (0, 0))
    pk_spec = pl.BlockSpec((bl, d // 2), lambda i: (i, 0))
    return pl.pallas_call(
        _mlp_both_block,
        grid=(n // bl,),
        in_specs=[row_spec, w_spec, b_spec, row_spec, w_spec, b_spec],
        out_specs=(pk_spec, pk_spec),
        out_shape=(jax.ShapeDtypeStruct((n, d // 2), jnp.int32),
                   jax.ShapeDtypeStruct((n, d // 2), jnp.int32)),
    )(eu, w1, b1.reshape(1, d), ev, w2, b2.reshape(1, d))


def _edge_values(u_tab, vneg_tab, edge_flat, ev):
    mesh = plsc.VectorSubcoreMesh(core_axis_name="c", subcore_axis_name="s")

    @functools.partial(
        pl.kernel,
        mesh=mesh,
        out_type=jax.ShapeDtypeStruct((_E,), jnp.float32),
        compiler_params=pltpu.CompilerParams(
            needs_layout_passes=False, use_tc_tiling_on_sc=False),
        scratch_types=[
            pltpu.VMEM((_EPW,), jnp.int32),
            pltpu.VMEM((_EPW,), jnp.int32),
            pltpu.VMEM((_EPW,), jnp.float32),
            pltpu.VMEM((_EPW,), jnp.float32),
            pltpu.VMEM((3, _G, _D // 2), jnp.int32),
            pltpu.VMEM((3, _G, _D // 2), jnp.int32),
            pltpu.SemaphoreType.DMA,
            pltpu.SemaphoreType.DMA,
            pltpu.SemaphoreType.DMA,
            pltpu.SemaphoreType.DMA,
            pltpu.SemaphoreType.DMA,
            pltpu.SemaphoreType.DMA,
        ],
    )
    def body(u_hbm, v_hbm, edge_hbm, ev_hbm, out_hbm,
             src_v, dst_v, ev_v, out_v, u_rows, v_rows,
             su0, su1, su2, sv0, sv1, sv2):
        wid = lax.axis_index("s") * 2 + lax.axis_index("c")
        base = wid * _EPW
        pltpu.sync_copy(edge_hbm.at[pl.ds(base, _EPW)], src_v)

        lanes = lax.iota(jnp.int32, 16)
        sem_u = (su0, su1, su2)
        sem_v = (sv0, sv1, sv2)

        def issue(g, b, n=_G):
            pltpu.async_copy(u_hbm.at[src_v.at[pl.ds(g * _G, n)]],
                             u_rows.at[b].at[pl.ds(0, n)], sem_u[b])
            pltpu.async_copy(v_hbm.at[dst_v.at[pl.ds(g * _G, n)]],
                             v_rows.at[b].at[pl.ds(0, n)], sem_v[b])

        def wait(g, b, n=_G):
            pltpu.make_async_copy(u_hbm.at[src_v.at[pl.ds(g * _G, n)]],
                                  u_rows.at[b].at[pl.ds(0, n)],
                                  sem_u[b]).wait()
            pltpu.make_async_copy(v_hbm.at[dst_v.at[pl.ds(g * _G, n)]],
                                  v_rows.at[b].at[pl.ds(0, n)],
                                  sem_v[b]).wait()

        def sub16(b, eb, sgb):
            d2 = jnp.zeros((16,), jnp.float32)
            for k in range(16):
                e = sgb + k
                acc = jnp.zeros((16,), jnp.float32)
                for j in range(_D // 32):
                    uw = u_rows[b, e, pl.ds(j * 16, 16)]
                    vw = v_rows[b, e, pl.ds(j * 16, 16)]
                    du = (plsc.bitcast(uw, jnp.bfloat16)
                          - plsc.bitcast(vw, jnp.bfloat16))
                    lo, hi = plsc.unpack(
                        du, format=plsc.PackFormat.INTERLEAVED)
                    acc = acc + lo * lo + hi * hi
                d2 = jnp.where(lanes == k, jnp.sum(acc), d2)
            d2c = jnp.maximum(d2, 1e-30)
            bi = lax.bitcast_convert_type(d2c, jnp.int32)
            bi = 0x5F3759DF - lax.shift_right_arithmetic(bi, 1)
            y = lax.bitcast_convert_type(bi, jnp.float32)
            for _ in range(3):
                y = y * (1.5 - 0.5 * d2c * y * y)
            dist = d2 * y
            sim = jnp.exp(dist)
            sig = 1.0 / (1.0 + jnp.exp(-sim))
            out_v[pl.ds(eb, 16)] = ev_v[pl.ds(eb, 16)] * sig

        def compute(g, b):
            gb = g * _G

            def subgroup(sg, c):
                sub16(b, gb + sg * 16, sg * 16)
                return c

            lax.fori_loop(0, _G // 16, subgroup, 0)

        pltpu.sync_copy(edge_hbm.at[pl.ds(_E + base, _EPW)], dst_v)
        issue(0, 0)
        pltpu.sync_copy(ev_hbm.at[pl.ds(base, _EPW)], ev_v)
        issue(1, 1)

        def outer(tt, carry):
            g0 = tt * 3
            for k in range(3):
                g = g0 + k
                issue(g + 2, (k + 2) % 3)
                wait(g, k)
                compute(g, k)
            return carry

        lax.fori_loop(0, (_NGF - 2) // 3, outer, 0)
        issue(_NGF, 2, n=16)
        wait(_NGF - 2, 0)
        compute(_NGF - 2, 0)
        wait(_NGF - 1, 1)
        compute(_NGF - 1, 1)
        wait(_NGF, 2, n=16)
        sub16(2, _TB, 0)

        pltpu.sync_copy(out_v, out_hbm.at[pl.ds(base, _EPW)])

    return body(u_tab, vneg_tab, edge_flat, ev)


def kernel(Eu, Ev, W1, b1, W2, b2, edge_index, edge_val):
    u, v = _transform_both(Eu, W1, b1, Ev, W2, b2)
    return _edge_values(u, v, edge_index.reshape(2 * _E), edge_val)
